# baseline (speedup 1.0000x reference)
import functools

import jax
import jax.numpy as jnp
from jax import lax
from jax.experimental import pallas as pl
from jax.experimental.pallas import tpu as pltpu
from jax.experimental.pallas import tpu_sc as plsc

N_FEATURES = 26
FEATURE_SIZE = 100000
EMB_DIM = 32
BATCH = 16384
HB = BATCH // 2

_INFO = plsc.get_sparse_core_info()
NC = _INFO.num_cores
NS = _INFO.num_subcores


@functools.partial(
    pl.kernel,
    mesh=plsc.VectorSubcoreMesh(core_axis_name="c", subcore_axis_name="s"),
    out_type=jax.ShapeDtypeStruct((N_FEATURES, EMB_DIM, BATCH), jnp.float32),
    scratch_types=[
        pltpu.VMEM((FEATURE_SIZE,), jnp.float32),
        pltpu.VMEM((HB,), jnp.int32),
        pltpu.VMEM((HB,), jnp.float32),
    ],
    compiler_params=pltpu.CompilerParams(
        use_tc_tiling_on_sc=False, needs_layout_passes=False
    ),
)
def _lookup_kernel(ids_hbm, table_hbm, out_hbm, col_v, ids_v, val_v):
    c = lax.axis_index("s") * NC + lax.axis_index("c")

    def per_feature(t, carry):
        pltpu.sync_copy(table_hbm.at[c, pl.ds(t * FEATURE_SIZE, FEATURE_SIZE)],
                        col_v)
        for h in range(BATCH // HB):
            i0 = h * HB
            pltpu.sync_copy(ids_hbm.at[t, pl.ds(i0, HB)], ids_v)

            def vec(k, inner):
                sl = pl.ds(k * 16, 16)
                val_v[sl] = plsc.load_gather(col_v, [ids_v[sl]])
                return inner

            lax.fori_loop(0, HB // 16, vec, 0)
            pltpu.sync_copy(val_v, out_hbm.at[t, c, pl.ds(i0, HB)])
        return carry

    lax.fori_loop(0, N_FEATURES, per_feature, 0)


def kernel(feature_ids, embed_weight):
    out = _lookup_kernel(feature_ids.T, embed_weight.T)
    return out.transpose(2, 0, 1)


# speedup vs baseline: 5.3996x; 5.3996x over previous
import functools

import jax
import jax.numpy as jnp
from jax import lax
from jax.experimental import pallas as pl
from jax.experimental.pallas import tpu as pltpu
from jax.experimental.pallas import tpu_sc as plsc

N_FEATURES = 26
FEATURE_SIZE = 100000
EMB_DIM = 32
BATCH = 16384
HB = BATCH // 2
TW = 100096
TABLE_COLS = N_FEATURES * FEATURE_SIZE

_INFO = plsc.get_sparse_core_info()
NC = _INFO.num_cores
NS = _INFO.num_subcores


TAIL = TABLE_COLS // 128 * 128
C0MAX = TAIL - TW


@functools.partial(
    pl.kernel,
    mesh=plsc.VectorSubcoreMesh(core_axis_name="c", subcore_axis_name="s"),
    out_type=jax.ShapeDtypeStruct((N_FEATURES, EMB_DIM, BATCH), jnp.float32),
    scratch_types=[
        pltpu.VMEM_SHARED((8, 12544), jnp.float32),
        pltpu.VMEM_SHARED((8, HB), jnp.float32),
        pltpu.VMEM((TW,), jnp.float32),
        pltpu.VMEM((HB // 128, 128), jnp.int32),
        pltpu.VMEM((HB,), jnp.float32),
        pltpu.VMEM((EMB_DIM, 128), jnp.float32),
    ],
    compiler_params=pltpu.CompilerParams(
        use_tc_tiling_on_sc=True, needs_layout_passes=False
    ),
)
def _lookup_kernel(ids_hbm, table_hbm, tail_hbm, out_hbm,
                   sp_tab, sp_out, col_v, ids_v, val_v, tail_v):
    cid = lax.axis_index("c")
    sid = lax.axis_index("s")
    srow = lax.rem(sid, 8)
    hh0 = (sid // 8) * (HB // 128)
    pltpu.sync_copy(tail_hbm, tail_v)

    def per_feature(t, carry):
        c0 = jnp.minimum(t * FEATURE_SIZE // 128 * 128, C0MAX)
        loff = t * FEATURE_SIZE - c0
        tail_lo = TAIL - t * FEATURE_SIZE
        pltpu.sync_copy(
            ids_hbm.at[pl.ds(pl.multiple_of(t * 128 + hh0, 8), HB // 128)],
            ids_v)

        for g in range(2):
            row0 = pl.multiple_of(cid * 16 + g * 8, 8)
            col = cid * 16 + g * 8 + srow

            for q0 in range(0, TW, 12544):
                qw = min(12544, TW - q0)

                @pl.when(sid == 0)
                def _stage_tab():
                    pltpu.sync_copy(
                        table_hbm.at[pl.ds(row0, 8),
                                     pl.ds(pl.multiple_of(c0 + q0, 128), qw)],
                        sp_tab.at[pl.ds(0, 8), pl.ds(0, qw)])

                plsc.subcore_barrier()
                pltpu.sync_copy(sp_tab.at[srow, pl.ds(0, qw)],
                                col_v.at[pl.ds(q0, qw)])
                plsc.subcore_barrier()

            def vec(k, inner):
                sl = pl.ds(k * 16, 16)
                ids16 = ids_v[k // 8, pl.ds((k % 8) * 16, 16)]
                v = plsc.load_gather(
                    col_v, [jnp.minimum(ids16 + loff, TW - 1)])
                tloc = jnp.maximum(ids16 - tail_lo, 0)
                tv = plsc.load_gather(
                    tail_v, [jnp.broadcast_to(col, (16,)), tloc])
                v = jnp.where(ids16 >= tail_lo, tv, v)
                val_v[sl] = v
                return inner

            lax.fori_loop(0, HB // 16, vec, 0)
            for hh in range(2):
                @pl.when(sid // 8 == hh)
                def _push():
                    pltpu.sync_copy(val_v, sp_out.at[srow])

                plsc.subcore_barrier()

                @pl.when(sid == 0)
                def _flush():
                    pltpu.sync_copy(
                        sp_out,
                        out_hbm.at[t, pl.ds(row0, 8), pl.ds(hh * HB, HB)])

                plsc.subcore_barrier()
        return carry

    lax.fori_loop(0, N_FEATURES, per_feature, 0)


def kernel(feature_ids, embed_weight):
    tail = jnp.pad(embed_weight[TAIL:].T, ((0, 0), (0, 128 - (TABLE_COLS - TAIL))))
    ids3 = feature_ids.T.reshape(N_FEATURES * 128, 128)
    out = _lookup_kernel(ids3, embed_weight.T, tail)
    return out.transpose(2, 0, 1)
